# baseline (speedup 1.0000x reference)
import math

import jax
import jax.numpy as jnp
from jax.experimental import pallas as pl
from jax.experimental.pallas import tpu as pltpu

_PS = 16
_B = 64
_N = 1024
_PD = 256
_DIM = 256
_MAXM = math.ceil(0.15 * _N)


def _prep_body(rand_ref, rpp_ref, rep_ref, rp_ref, pos_ref, wt_ref, bt_ref,
               wb_ref, bb_ref, wc_ref, c_ref, src_ref):
    wc = jax.lax.dot(wt_ref[...], wb_ref[...], preferred_element_type=jnp.float32)
    wc_ref[...] = wc
    base = jax.lax.dot(bt_ref[...], wb_ref[...], preferred_element_type=jnp.float32) + bb_ref[...]
    c_ref[...] = jax.lax.dot(pos_ref[...], wc, preferred_element_type=jnp.float32) + base

    r = rand_ref[...]

    def bs(_, carry):
        lo, hi = carry
        mid = 0.5 * (lo + hi)
        cnt = jnp.sum((r > mid).astype(jnp.float32), axis=1, keepdims=True)
        ge = cnt >= _MAXM
        return jnp.where(ge, mid, lo), jnp.where(ge, hi, mid)

    lo, _ = jax.lax.fori_loop(
        0, 30, bs, (jnp.zeros((_B, 1), jnp.float32), jnp.ones((_B, 1), jnp.float32)))
    maskb = r > lo
    bmr = maskb & (rep_ref[...] != 0)
    brp = maskb & (rpp_ref[...] != 0)
    iot = jax.lax.broadcasted_iota(jnp.int32, (_B, _N), 1)
    src = jnp.where(brp, rp_ref[...], iot)
    src_ref[...] = jnp.where(bmr, _N, src)


_BPG = 2


def _main_body(p_ref, src_ref, tok_ref,
               s1_ref, b1_ref, we_ref, be_ref, s2_ref, b2_ref,
               wc_ref, c_ref, acc_ref):
    g = pl.program_id(0)
    i0 = jax.lax.broadcasted_iota(jnp.int32, (_N + 1, _N), 0)
    ssq = jnp.zeros((1, 1), jnp.float32)
    for u in range(_BPG):
        P = p_ref[u]
        src2 = src_ref[pl.ds(g * _BPG + u, 1), :]
        P_ext = jnp.concatenate([P, tok_ref[...]], axis=0)
        St = (i0 == src2).astype(jnp.float32)
        masked = jax.lax.dot_general(St, P_ext, (((0,), (0,)), ((), ())),
                                     preferred_element_type=jnp.float32)

        mu = jnp.mean(masked, axis=1, keepdims=True)
        xm = masked - mu
        var = jnp.mean(xm * xm, axis=1, keepdims=True)
        xh = xm * jax.lax.rsqrt(var + 1e-5) * s1_ref[...] + b1_ref[...]
        x = jax.lax.dot(xh, we_ref[...], preferred_element_type=jnp.float32) + be_ref[...]
        mu2 = jnp.mean(x, axis=1, keepdims=True)
        xm2 = x - mu2
        var2 = jnp.mean(xm2 * xm2, axis=1, keepdims=True)
        xe = xm2 * jax.lax.rsqrt(var2 + 1e-5) * s2_ref[...] + b2_ref[...]
        resid = jax.lax.dot(xe, wc_ref[...], preferred_element_type=jnp.float32) + c_ref[...] - P
        ssq = ssq + jnp.reshape(jnp.sum(resid * resid), (1, 1))

    prev = jnp.where(g == 0, jnp.zeros((1, 1), jnp.float32), acc_ref[...])
    tot = prev + ssq
    acc_ref[...] = jnp.where(g == _B // _BPG - 1, tot * (1.0 / (_B * _N * _PD)), tot)


def kernel(input, mask_token, ln1_s, ln1_b, W_embed, b_embed, ln2_s, ln2_b,
           cls_token, pos_embedding, W_t, b_t, W_bits, b_bits):
    B, H, W = input.shape
    hh, ww = H // _PS, W // _PS
    n = hh * ww

    patches = input.reshape(B, hh, _PS, ww, _PS).transpose(0, 1, 3, 2, 4).reshape(B, n, _PS * _PS)

    mk = jax.random.key(1)
    k1, k2, k3, k4 = jax.random.split(mk, 4)
    rand = jax.random.uniform(k1, (B, n))
    rps_prob = 0.5 / (1.0 - 0.5)
    rpp = (jax.random.uniform(k2, (B, n)) < rps_prob).astype(jnp.int32)
    rp = jax.random.randint(k3, (B, n), 0, n).astype(jnp.int32)
    rep = (jax.random.uniform(k4, (B, n)) < 0.5).astype(jnp.int32)

    pos_rows = pos_embedding[0, 1:n + 1, :]
    bt2 = b_t.reshape(1, _DIM)
    bb2 = b_bits.reshape(1, _PD)
    wc, c_rows, src = pl.pallas_call(
        _prep_body,
        out_shape=(jax.ShapeDtypeStruct((_DIM, _PD), jnp.float32),
                   jax.ShapeDtypeStruct((n, _PD), jnp.float32),
                   jax.ShapeDtypeStruct((B, n), jnp.int32)),
    )(rand, rpp, rep, rp, pos_rows, W_t, bt2, W_bits, bb2)

    tok = mask_token.reshape(1, _PD)
    s1 = ln1_s.reshape(1, _PD)
    b1 = ln1_b.reshape(1, _PD)
    be = b_embed.reshape(1, _DIM)
    s2 = ln2_s.reshape(1, _DIM)
    b2 = ln2_b.reshape(1, _DIM)

    full = lambda shape: pl.BlockSpec(shape, lambda b: tuple(0 for _ in shape))
    acc = pl.pallas_call(
        _main_body,
        grid=(B // _BPG,),
        in_specs=[
            pl.BlockSpec((_BPG, n, _PD), lambda g: (g, 0, 0)),
            full((B, n)),
            full((1, _PD)), full((1, _PD)), full((1, _PD)),
            full((_PD, _DIM)), full((1, _DIM)), full((1, _DIM)), full((1, _DIM)),
            full((_DIM, _PD)), full((n, _PD)),
        ],
        out_specs=pl.BlockSpec((1, 1), lambda b: (0, 0)),
        out_shape=jax.ShapeDtypeStruct((1, 1), jnp.float32),
        compiler_params=pltpu.CompilerParams(
            dimension_semantics=("arbitrary",)),
    )(patches, src, tok,
      s1, b1, W_embed, be, s2, b2, wc, c_rows)
    return acc[0, 0]


# speedup vs baseline: 1.0157x; 1.0157x over previous
import math

import jax
import jax.numpy as jnp
from jax.experimental import pallas as pl
from jax.experimental.pallas import tpu as pltpu

_PS = 16
_B = 64
_N = 1024
_PD = 256
_DIM = 256
_MAXM = math.ceil(0.15 * _N)


def _prep_body(rand_ref, rpp_ref, rep_ref, rp_ref, pos_ref, wt_ref, bt_ref,
               wb_ref, bb_ref, wc_ref, c_ref, src_ref):
    wc = jax.lax.dot(wt_ref[...], wb_ref[...], preferred_element_type=jnp.float32)
    wc_ref[...] = wc
    base = jax.lax.dot(bt_ref[...], wb_ref[...], preferred_element_type=jnp.float32) + bb_ref[...]
    c_ref[...] = jax.lax.dot(pos_ref[...], wc, preferred_element_type=jnp.float32) + base

    r = rand_ref[...]

    def bs(_, carry):
        lo, hi = carry
        mid = 0.5 * (lo + hi)
        cnt = jnp.sum((r > mid).astype(jnp.float32), axis=1, keepdims=True)
        ge = cnt >= _MAXM
        return jnp.where(ge, mid, lo), jnp.where(ge, hi, mid)

    lo, _ = jax.lax.fori_loop(
        0, 30, bs, (jnp.zeros((_B, 1), jnp.float32), jnp.ones((_B, 1), jnp.float32)))
    maskb = r > lo
    bmr = maskb & (rep_ref[...] != 0)
    brp = maskb & (rpp_ref[...] != 0)
    iot = jax.lax.broadcasted_iota(jnp.int32, (_B, _N), 1)
    src = jnp.where(brp, rp_ref[...], iot)
    src_ref[...] = jnp.where(bmr, _N, src)


def _main_body(p_ref, src_ref, tok_ref,
               s1_ref, b1_ref, we_ref, be_ref, s2_ref, b2_ref,
               wc_ref, c_ref, acc_ref):
    b = pl.program_id(0)
    P = p_ref[0]
    src2 = src_ref[pl.ds(b, 1), :]
    P_ext = jnp.concatenate([P, tok_ref[...]], axis=0)
    i0 = jax.lax.broadcasted_iota(jnp.int32, (_N + 1, _N), 0)
    St = (i0 == src2).astype(jnp.float32)
    masked = jax.lax.dot_general(St, P_ext, (((0,), (0,)), ((), ())),
                                 preferred_element_type=jnp.float32)

    mu = jnp.mean(masked, axis=1, keepdims=True)
    xm = masked - mu
    var = jnp.mean(xm * xm, axis=1, keepdims=True)
    xh = xm * jax.lax.rsqrt(var + 1e-5) * s1_ref[...] + b1_ref[...]
    x = jax.lax.dot(xh, we_ref[...], preferred_element_type=jnp.float32) + be_ref[...]
    mu2 = jnp.mean(x, axis=1, keepdims=True)
    xm2 = x - mu2
    var2 = jnp.mean(xm2 * xm2, axis=1, keepdims=True)
    xe = xm2 * jax.lax.rsqrt(var2 + 1e-5) * s2_ref[...] + b2_ref[...]
    resid = jax.lax.dot(xe, wc_ref[...], preferred_element_type=jnp.float32) + c_ref[...] - P
    ssq = jnp.reshape(jnp.sum(resid * resid), (1, 1))

    prev = jnp.where(b == 0, jnp.zeros((1, 1), jnp.float32), acc_ref[...])
    tot = prev + ssq
    acc_ref[...] = jnp.where(b == _B - 1, tot * (1.0 / (_B * _N * _PD)), tot)


def kernel(input, mask_token, ln1_s, ln1_b, W_embed, b_embed, ln2_s, ln2_b,
           cls_token, pos_embedding, W_t, b_t, W_bits, b_bits):
    B, H, W = input.shape
    hh, ww = H // _PS, W // _PS
    n = hh * ww

    patches = input.reshape(B, hh, _PS, ww, _PS).transpose(0, 1, 3, 2, 4).reshape(B, n, _PS * _PS)

    mk = jax.random.key(1)
    k1, k2, k3, k4 = jax.random.split(mk, 4)
    rand = jax.random.uniform(k1, (B, n))
    rps_prob = 0.5 / (1.0 - 0.5)
    rpp = (jax.random.uniform(k2, (B, n)) < rps_prob).astype(jnp.int32)
    rp = jax.random.randint(k3, (B, n), 0, n).astype(jnp.int32)
    rep = (jax.random.uniform(k4, (B, n)) < 0.5).astype(jnp.int32)

    pos_rows = pos_embedding[0, 1:n + 1, :]
    bt2 = b_t.reshape(1, _DIM)
    bb2 = b_bits.reshape(1, _PD)
    wc, c_rows, src = pl.pallas_call(
        _prep_body,
        out_shape=(jax.ShapeDtypeStruct((_DIM, _PD), jnp.float32),
                   jax.ShapeDtypeStruct((n, _PD), jnp.float32),
                   jax.ShapeDtypeStruct((B, n), jnp.int32)),
    )(rand, rpp, rep, rp, pos_rows, W_t, bt2, W_bits, bb2)

    tok = mask_token.reshape(1, _PD)
    s1 = ln1_s.reshape(1, _PD)
    b1 = ln1_b.reshape(1, _PD)
    be = b_embed.reshape(1, _DIM)
    s2 = ln2_s.reshape(1, _DIM)
    b2 = ln2_b.reshape(1, _DIM)

    full = lambda shape: pl.BlockSpec(shape, lambda b: tuple(0 for _ in shape))
    acc = pl.pallas_call(
        _main_body,
        grid=(B,),
        in_specs=[
            pl.BlockSpec((1, n, _PD), lambda b: (b, 0, 0)),
            full((B, n)),
            full((1, _PD)), full((1, _PD)), full((1, _PD)),
            full((_PD, _DIM)), full((1, _DIM)), full((1, _DIM)), full((1, _DIM)),
            full((_DIM, _PD)), full((n, _PD)),
        ],
        out_specs=pl.BlockSpec((1, 1), lambda b: (0, 0)),
        out_shape=jax.ShapeDtypeStruct((1, 1), jnp.float32),
        compiler_params=pltpu.CompilerParams(
            dimension_semantics=("arbitrary",)),
    )(patches, src, tok,
      s1, b1, W_embed, be, s2, b2, wc, c_rows)
    return acc[0, 0]
